# baseline (speedup 1.0000x reference)
import functools

import jax
import jax.numpy as jnp
from jax import lax
from jax.experimental import pallas as pl
from jax.experimental.pallas import tpu as pltpu
from jax.experimental.pallas import tpu_sc as plsc

N = 10000
E = 320000
D = 128
G = 64
OUT = 16

NS = 16
CB = 128
BK = 16
NB = 10
CH = BK * NB
EP = NS * CH * CB
NA = 10240
RPT = NA // NS

_mesh = plsc.VectorSubcoreMesh(core_axis_name="c", subcore_axis_name="s",
                               num_cores=1)


@functools.partial(
    pl.kernel,
    mesh=_mesh,
    out_type=jax.ShapeDtypeStruct((NA, D), jnp.float32),
    scratch_types=[
        pltpu.VMEM((BK, CB), jnp.int32),
        pltpu.VMEM((BK, CB), jnp.int32),
        pltpu.VMEM((CB, D), jnp.float32),
        pltpu.VMEM((CB, D), jnp.float32),
        pltpu.VMEM_SHARED((NA, D), jnp.float32),
        pltpu.SemaphoreType.DMA,
        pltpu.SemaphoreType.DMA,
    ],
)
def _sc_agg(h_hbm, src_hbm, dst_hbm, out_hbm, src_blk, dst_blk,
            rows0, rows1, acc_sh, sem0, sem1):
    s = lax.axis_index("s")

    zeros16 = jnp.zeros((16,), jnp.float32)

    def _zero_row(i, carry):
        for k in range(D // 16):
            rows0[i, pl.ds(k * 16, 16)] = zeros16
        return carry

    lax.fori_loop(0, CB, _zero_row, 0)
    for t in range(RPT // CB):
        pltpu.sync_copy(rows0, acc_sh.at[pl.ds(s * RPT + t * CB, CB)])
    plsc.subcore_barrier()

    def _block(b, carry):
        pltpu.sync_copy(src_hbm.at[s, pl.ds(b * BK, BK)], src_blk)
        pltpu.sync_copy(dst_hbm.at[s, pl.ds(b * BK, BK)], dst_blk)
        pltpu.async_copy(h_hbm.at[src_blk.at[0]], rows0, sem0)

        def _pair(t, carry2):
            j = 2 * t
            pltpu.async_copy(h_hbm.at[src_blk.at[j + 1]], rows1, sem1)
            pltpu.make_async_copy(h_hbm.at[src_blk.at[j]], rows0, sem0).wait()
            pltpu.sync_copy(rows0, acc_sh.at[dst_blk.at[j]], add=True)

            @pl.when(j + 2 < BK)
            def _():
                pltpu.async_copy(h_hbm.at[src_blk.at[j + 2]], rows0, sem0)

            pltpu.make_async_copy(h_hbm.at[src_blk.at[j + 1]], rows1,
                                  sem1).wait()
            pltpu.sync_copy(rows1, acc_sh.at[dst_blk.at[j + 1]], add=True)
            return carry2

        lax.fori_loop(0, BK // 2, _pair, 0)
        return carry

    lax.fori_loop(0, NB, _block, 0)

    plsc.subcore_barrier()
    pltpu.sync_copy(acc_sh.at[pl.ds(s * RPT, RPT)],
                    out_hbm.at[pl.ds(s * RPT, RPT)])


def _mlp_body(h_ref, agg_ref, wa_ref, ba_ref, ga_ref, bea_ref,
              wb_ref, bb_ref, gb_ref, beb_ref, batch_ref,
              h_out, pool_out):
    z = h_ref[...] + agg_ref[:N, :]

    def _lin_bn_relu(v, w_ref, b_ref, g_ref, be_ref):
        y = jnp.dot(v, w_ref[...], preferred_element_type=jnp.float32)
        y = y + b_ref[...]
        m = jnp.mean(y, axis=0, keepdims=True)
        var = jnp.mean(y * y, axis=0, keepdims=True) - m * m
        y = g_ref[...] * (y - m) * lax.rsqrt(var + 1e-5) + be_ref[...]
        return jnp.maximum(y, 0.0)

    y = _lin_bn_relu(z, wa_ref, ba_ref, ga_ref, bea_ref)
    y = _lin_bn_relu(y, wb_ref, bb_ref, gb_ref, beb_ref)
    h_out[...] = y

    seg = lax.broadcasted_iota(jnp.int32, (G, N), 0)
    onehot = jnp.where(seg == batch_ref[...], 1.0, 0.0)
    pool_out[...] = jnp.dot(onehot, y, preferred_element_type=jnp.float32)


_mlp = pl.pallas_call(
    _mlp_body,
    out_shape=(jax.ShapeDtypeStruct((N, D), jnp.float32),
               jax.ShapeDtypeStruct((G, D), jnp.float32)),
)


def _head_body(x_ref, batch_ref, p1, p2, p3, p4, p5,
               fc1w_ref, fc1b_ref, fc3w_ref, fc3b_ref, out_ref):
    seg = lax.broadcasted_iota(jnp.int32, (G, N), 0)
    onehot = jnp.where(seg == batch_ref[...], 1.0, 0.0)
    px = jnp.dot(onehot, x_ref[...], preferred_element_type=jnp.float32)
    hg = jnp.concatenate(
        [px, p1[...], p2[...], p3[...], p4[...], p5[...]], axis=1)
    r = jnp.dot(hg, fc1w_ref[...], preferred_element_type=jnp.float32)
    r = jnp.maximum(r + fc1b_ref[...], 0.0)
    o = jnp.dot(r, fc3w_ref[...], preferred_element_type=jnp.float32)
    out_ref[...] = o + fc3b_ref[...]


_head = pl.pallas_call(
    _head_body,
    out_shape=jax.ShapeDtypeStruct((G, OUT), jnp.float32),
)


def kernel(x, edge_index, batch, Wa, ba, ga, bea, Wb, bb, gb, beb,
           fc1_W, fc1_b, fc3_W, fc3_b):
    src = edge_index[0].astype(jnp.int32)
    dst = edge_index[1].astype(jnp.int32)
    pad = EP - E
    src_p = jnp.concatenate([src, jnp.zeros((pad,), jnp.int32)])
    dst_p = jnp.concatenate([dst, jnp.full((pad,), N, jnp.int32)])
    src_r = src_p.reshape(NS, CH, CB)
    dst_r = dst_p.reshape(NS, CH, CB)
    batch_r = batch.astype(jnp.int32).reshape(1, N)

    h = x
    pooled = []
    for i in range(5):
        agg = _sc_agg(h, src_r, dst_r)
        h, p = _mlp(h, agg, Wa[i],
                    ba[i].reshape(1, D), ga[i].reshape(1, D),
                    bea[i].reshape(1, D), Wb[i],
                    bb[i].reshape(1, D), gb[i].reshape(1, D),
                    beb[i].reshape(1, D), batch_r)
        pooled.append(p)

    return _head(x, batch_r, *pooled,
                 fc1_W, fc1_b.reshape(1, 6 * D), fc3_W, fc3_b.reshape(1, OUT))


# speedup vs baseline: 2.0385x; 2.0385x over previous
import functools

import jax
import jax.numpy as jnp
from jax import lax
from jax.experimental import pallas as pl
from jax.experimental.pallas import tpu as pltpu
from jax.experimental.pallas import tpu_sc as plsc

N = 10000
E = 320000
D = 128
G = 64
OUT = 16

NS = 16
CB = 128
BK = 16
NB = 10
CH = BK * NB
EP = NS * CH * CB
NA = 10240
RPT = NA // NS

_mesh = plsc.VectorSubcoreMesh(core_axis_name="c", subcore_axis_name="s",
                               num_cores=1)


@functools.partial(
    pl.kernel,
    mesh=_mesh,
    out_type=jax.ShapeDtypeStruct((NA, D), jnp.float32),
    scratch_types=[
        pltpu.VMEM((BK, CB), jnp.int32),
        pltpu.VMEM((BK, CB), jnp.int32),
        pltpu.VMEM((2, CB, D), jnp.float32),
        pltpu.VMEM_SHARED((NA, D), jnp.float32),
        pltpu.SemaphoreType.DMA,
        pltpu.SemaphoreType.DMA,
        pltpu.SemaphoreType.DMA,
        pltpu.SemaphoreType.DMA,
    ],
)
def _sc_agg(h_hbm, src_hbm, dst_hbm, out_hbm, src_blk, dst_blk, rows,
            acc_sh, g0, g1, s0, s1):
    s = lax.axis_index("s")
    gsem = [g0, g1]
    ssem = [s0, s1]

    zeros16 = jnp.zeros((16,), jnp.float32)

    def _zero_row(i, carry):
        for k in range(D // 16):
            rows[0, i, pl.ds(k * 16, 16)] = zeros16
        return carry

    lax.fori_loop(0, CB, _zero_row, 0)
    for t in range(RPT // CB):
        pltpu.sync_copy(rows.at[0], acc_sh.at[pl.ds(s * RPT + t * CB, CB)])
    plsc.subcore_barrier()

    def _gather(jl, b):
        pltpu.async_copy(h_hbm.at[src_blk.at[jl]], rows.at[b], gsem[b])

    def _wait_gather(jl, b):
        pltpu.make_async_copy(h_hbm.at[src_blk.at[jl]], rows.at[b],
                              gsem[b]).wait()

    def _scatter(jl, b):
        pltpu.async_copy(rows.at[b], acc_sh.at[dst_blk.at[jl]], ssem[b],
                         add=True)

    def _wait_scatter(jl, b):
        pltpu.make_async_copy(rows.at[b], acc_sh.at[dst_blk.at[jl]],
                              ssem[b]).wait()

    def _block(kb, carry):
        @pl.when(kb > 0)
        def _():
            _wait_scatter(BK - 2, 0)
            _wait_scatter(BK - 1, 1)

        pltpu.sync_copy(src_hbm.at[s, pl.ds(kb * BK, BK)], src_blk)
        pltpu.sync_copy(dst_hbm.at[s, pl.ds(kb * BK, BK)], dst_blk)

        _gather(0, 0)
        for jl in range(BK - 1):
            b = jl % 2
            _wait_gather(jl, b)
            _scatter(jl, b)
            if jl > 0:
                _wait_scatter(jl - 1, 1 - b)
            _gather(jl + 1, 1 - b)
        _wait_gather(BK - 1, 1)
        _scatter(BK - 1, 1)
        return carry

    lax.fori_loop(0, NB, _block, 0)
    _wait_scatter(BK - 2, 0)
    _wait_scatter(BK - 1, 1)

    plsc.subcore_barrier()
    pltpu.sync_copy(acc_sh.at[pl.ds(s * RPT, RPT)],
                    out_hbm.at[pl.ds(s * RPT, RPT)])


def _mlp_body(h_ref, agg_ref, wa_ref, ba_ref, ga_ref, bea_ref,
              wb_ref, bb_ref, gb_ref, beb_ref, batch_ref,
              h_out, pool_out):
    z = h_ref[...] + agg_ref[:N, :]

    def _lin_bn_relu(v, w_ref, b_ref, g_ref, be_ref):
        y = jnp.dot(v, w_ref[...], preferred_element_type=jnp.float32)
        y = y + b_ref[...]
        m = jnp.mean(y, axis=0, keepdims=True)
        var = jnp.mean(y * y, axis=0, keepdims=True) - m * m
        y = g_ref[...] * (y - m) * lax.rsqrt(var + 1e-5) + be_ref[...]
        return jnp.maximum(y, 0.0)

    y = _lin_bn_relu(z, wa_ref, ba_ref, ga_ref, bea_ref)
    y = _lin_bn_relu(y, wb_ref, bb_ref, gb_ref, beb_ref)
    h_out[...] = y

    seg = lax.broadcasted_iota(jnp.int32, (G, N), 0)
    onehot = jnp.where(seg == batch_ref[...], 1.0, 0.0)
    pool_out[...] = jnp.dot(onehot, y, preferred_element_type=jnp.float32)


_mlp = pl.pallas_call(
    _mlp_body,
    out_shape=(jax.ShapeDtypeStruct((N, D), jnp.float32),
               jax.ShapeDtypeStruct((G, D), jnp.float32)),
)


def _head_body(x_ref, batch_ref, p1, p2, p3, p4, p5,
               fc1w_ref, fc1b_ref, fc3w_ref, fc3b_ref, out_ref):
    seg = lax.broadcasted_iota(jnp.int32, (G, N), 0)
    onehot = jnp.where(seg == batch_ref[...], 1.0, 0.0)
    px = jnp.dot(onehot, x_ref[...], preferred_element_type=jnp.float32)
    hg = jnp.concatenate(
        [px, p1[...], p2[...], p3[...], p4[...], p5[...]], axis=1)
    r = jnp.dot(hg, fc1w_ref[...], preferred_element_type=jnp.float32)
    r = jnp.maximum(r + fc1b_ref[...], 0.0)
    o = jnp.dot(r, fc3w_ref[...], preferred_element_type=jnp.float32)
    out_ref[...] = o + fc3b_ref[...]


_head = pl.pallas_call(
    _head_body,
    out_shape=jax.ShapeDtypeStruct((G, OUT), jnp.float32),
)


def kernel(x, edge_index, batch, Wa, ba, ga, bea, Wb, bb, gb, beb,
           fc1_W, fc1_b, fc3_W, fc3_b):
    src = edge_index[0].astype(jnp.int32)
    dst = edge_index[1].astype(jnp.int32)
    pad = EP - E
    pad_i = lax.iota(jnp.int32, pad)
    src_p = jnp.concatenate([src, pad_i % N])
    dst_p = jnp.concatenate([dst, N + pad_i % (NA - N)])
    src_r = src_p.reshape(NS, CH, CB)
    dst_r = dst_p.reshape(NS, CH, CB)
    batch_r = batch.astype(jnp.int32).reshape(1, N)

    h = x
    pooled = []
    for i in range(5):
        agg = _sc_agg(h, src_r, dst_r)
        h, p = _mlp(h, agg, Wa[i],
                    ba[i].reshape(1, D), ga[i].reshape(1, D),
                    bea[i].reshape(1, D), Wb[i],
                    bb[i].reshape(1, D), gb[i].reshape(1, D),
                    beb[i].reshape(1, D), batch_r)
        pooled.append(p)

    return _head(x, batch_r, *pooled,
                 fc1_W, fc1_b.reshape(1, 6 * D), fc3_W, fc3_b.reshape(1, OUT))


# speedup vs baseline: 2.1229x; 1.0414x over previous
import functools

import jax
import jax.numpy as jnp
from jax import lax
from jax.experimental import pallas as pl
from jax.experimental.pallas import tpu as pltpu
from jax.experimental.pallas import tpu_sc as plsc

N = 10000
E = 320000
D = 128
G = 64
OUT = 16

NS = 16
CB = 128
BK = 16
NB = 10
CH = BK * NB
EP = NS * CH * CB
NA = 10240
RPT = NA // NS

_mesh = plsc.VectorSubcoreMesh(core_axis_name="c", subcore_axis_name="s",
                               num_cores=1)


@functools.partial(
    pl.kernel,
    mesh=_mesh,
    out_type=jax.ShapeDtypeStruct((NA, D), jnp.float32),
    scratch_types=[
        pltpu.VMEM((2, BK, CB), jnp.int32),
        pltpu.VMEM((2, BK, CB), jnp.int32),
        pltpu.VMEM((2, CB, D), jnp.float32),
        pltpu.VMEM_SHARED((NA, D), jnp.float32),
        pltpu.SemaphoreType.DMA,
        pltpu.SemaphoreType.DMA,
        pltpu.SemaphoreType.DMA,
        pltpu.SemaphoreType.DMA,
        pltpu.SemaphoreType.DMA,
        pltpu.SemaphoreType.DMA,
    ],
)
def _sc_agg(h_hbm, src_hbm, dst_hbm, out_hbm, src_blk, dst_blk, rows,
            acc_sh, g0, g1, s0, s1, isem_s, isem_d):
    s = lax.axis_index("s")
    gsem = [g0, g1]
    ssem = [s0, s1]

    zeros16 = jnp.zeros((16,), jnp.float32)

    def _zero_row(i, carry):
        for k in range(D // 16):
            rows[0, i, pl.ds(k * 16, 16)] = zeros16
        return carry

    lax.fori_loop(0, CB, _zero_row, 0)
    for t in range(RPT // CB):
        pltpu.sync_copy(rows.at[0], acc_sh.at[pl.ds(s * RPT + t * CB, CB)])

    def _gather(p, jl, b):
        pltpu.async_copy(h_hbm.at[src_blk.at[p, jl]], rows.at[b], gsem[b])

    def _wait_gather(b):
        pltpu.make_async_copy(h_hbm.at[src_blk.at[0, 0]], rows.at[b],
                              gsem[b]).wait()

    def _scatter(p, jl, b):
        pltpu.async_copy(rows.at[b], acc_sh.at[dst_blk.at[p, jl]], ssem[b],
                         add=True)

    def _wait_scatter(b):
        pltpu.make_async_copy(rows.at[b], acc_sh.at[dst_blk.at[0, 0]],
                              ssem[b]).wait()

    pltpu.sync_copy(src_hbm.at[s, pl.ds(0, BK)], src_blk.at[0])
    pltpu.sync_copy(dst_hbm.at[s, pl.ds(0, BK)], dst_blk.at[0])
    plsc.subcore_barrier()

    def _block(kb, carry):
        p = lax.rem(kb, 2)

        @pl.when(kb > 0)
        def _():
            _wait_scatter(0)
            _wait_scatter(1)

        @pl.when(kb + 1 < NB)
        def _():
            pltpu.async_copy(src_hbm.at[s, pl.ds((kb + 1) * BK, BK)],
                             src_blk.at[1 - p], isem_s)
            pltpu.async_copy(dst_hbm.at[s, pl.ds((kb + 1) * BK, BK)],
                             dst_blk.at[1 - p], isem_d)

        _gather(p, 0, 0)
        for jl in range(BK - 1):
            b = jl % 2
            _wait_gather(b)
            _scatter(p, jl, b)
            if jl > 0:
                _wait_scatter(1 - b)
            _gather(p, jl + 1, 1 - b)
        _wait_gather(1)
        _scatter(p, BK - 1, 1)

        @pl.when(kb + 1 < NB)
        def _():
            pltpu.make_async_copy(src_hbm.at[s, pl.ds(0, BK)],
                                  src_blk.at[0], isem_s).wait()
            pltpu.make_async_copy(dst_hbm.at[s, pl.ds(0, BK)],
                                  dst_blk.at[0], isem_d).wait()

        return carry

    lax.fori_loop(0, NB, _block, 0)
    _wait_scatter(0)
    _wait_scatter(1)

    plsc.subcore_barrier()
    pltpu.sync_copy(acc_sh.at[pl.ds(s * RPT, RPT)],
                    out_hbm.at[pl.ds(s * RPT, RPT)])


def _mlp_body(h_ref, agg_ref, wa_ref, ba_ref, ga_ref, bea_ref,
              wb_ref, bb_ref, gb_ref, beb_ref, batch_ref,
              h_out, pool_out):
    z = h_ref[...] + agg_ref[:N, :]

    def _lin_bn_relu(v, w_ref, b_ref, g_ref, be_ref):
        y = jnp.dot(v, w_ref[...], preferred_element_type=jnp.float32)
        y = y + b_ref[...]
        m = jnp.mean(y, axis=0, keepdims=True)
        var = jnp.mean(y * y, axis=0, keepdims=True) - m * m
        y = g_ref[...] * (y - m) * lax.rsqrt(var + 1e-5) + be_ref[...]
        return jnp.maximum(y, 0.0)

    y = _lin_bn_relu(z, wa_ref, ba_ref, ga_ref, bea_ref)
    y = _lin_bn_relu(y, wb_ref, bb_ref, gb_ref, beb_ref)
    h_out[...] = y

    seg = lax.broadcasted_iota(jnp.int32, (G, N), 0)
    onehot = jnp.where(seg == batch_ref[...], 1.0, 0.0)
    pool_out[...] = jnp.dot(onehot, y, preferred_element_type=jnp.float32)


_mlp = pl.pallas_call(
    _mlp_body,
    out_shape=(jax.ShapeDtypeStruct((N, D), jnp.float32),
               jax.ShapeDtypeStruct((G, D), jnp.float32)),
)


def _head_body(x_ref, batch_ref, p1, p2, p3, p4, p5,
               fc1w_ref, fc1b_ref, fc3w_ref, fc3b_ref, out_ref):
    seg = lax.broadcasted_iota(jnp.int32, (G, N), 0)
    onehot = jnp.where(seg == batch_ref[...], 1.0, 0.0)
    px = jnp.dot(onehot, x_ref[...], preferred_element_type=jnp.float32)
    hg = jnp.concatenate(
        [px, p1[...], p2[...], p3[...], p4[...], p5[...]], axis=1)
    r = jnp.dot(hg, fc1w_ref[...], preferred_element_type=jnp.float32)
    r = jnp.maximum(r + fc1b_ref[...], 0.0)
    o = jnp.dot(r, fc3w_ref[...], preferred_element_type=jnp.float32)
    out_ref[...] = o + fc3b_ref[...]


_head = pl.pallas_call(
    _head_body,
    out_shape=jax.ShapeDtypeStruct((G, OUT), jnp.float32),
)


def kernel(x, edge_index, batch, Wa, ba, ga, bea, Wb, bb, gb, beb,
           fc1_W, fc1_b, fc3_W, fc3_b):
    src = edge_index[0].astype(jnp.int32)
    dst = edge_index[1].astype(jnp.int32)
    pad = EP - E
    pad_i = lax.iota(jnp.int32, pad)
    src_p = jnp.concatenate([src, pad_i % N])
    dst_p = jnp.concatenate([dst, N + pad_i % (NA - N)])
    src_r = src_p.reshape(NS, CH, CB)
    dst_r = dst_p.reshape(NS, CH, CB)
    batch_r = batch.astype(jnp.int32).reshape(1, N)

    h = x
    pooled = []
    for i in range(5):
        agg = _sc_agg(h, src_r, dst_r)
        h, p = _mlp(h, agg, Wa[i],
                    ba[i].reshape(1, D), ga[i].reshape(1, D),
                    bea[i].reshape(1, D), Wb[i],
                    bb[i].reshape(1, D), gb[i].reshape(1, D),
                    beb[i].reshape(1, D), batch_r)
        pooled.append(p)

    return _head(x, batch_r, *pooled,
                 fc1_W, fc1_b.reshape(1, 6 * D), fc3_W, fc3_b.reshape(1, OUT))


# speedup vs baseline: 2.5183x; 1.1863x over previous
import functools

import jax
import jax.numpy as jnp
from jax import lax
from jax.experimental import pallas as pl
from jax.experimental.pallas import tpu as pltpu
from jax.experimental.pallas import tpu_sc as plsc

N = 10000
E = 320000
D = 128
G = 64
OUT = 16

NS = 16
CB = 64
NR = 4
BK = 32
NB = 10
CH = BK * NB
EP = NS * CH * CB
NA = 10240
RPT = NA // NS

_mesh = plsc.VectorSubcoreMesh(core_axis_name="c", subcore_axis_name="s",
                               num_cores=1)


@functools.partial(
    pl.kernel,
    mesh=_mesh,
    out_type=jax.ShapeDtypeStruct((NA, D), jnp.float32),
    scratch_types=[
        pltpu.VMEM((2, BK, CB), jnp.int32),
        pltpu.VMEM((2, BK, CB), jnp.int32),
        pltpu.VMEM((NR, CB, D), jnp.float32),
        pltpu.VMEM_SHARED((NA, D), jnp.float32),
        pltpu.SemaphoreType.DMA,
        pltpu.SemaphoreType.DMA,
        pltpu.SemaphoreType.DMA,
        pltpu.SemaphoreType.DMA,
        pltpu.SemaphoreType.DMA,
        pltpu.SemaphoreType.DMA,
        pltpu.SemaphoreType.DMA,
        pltpu.SemaphoreType.DMA,
        pltpu.SemaphoreType.DMA,
        pltpu.SemaphoreType.DMA,
    ],
)
def _sc_agg(h_hbm, src_hbm, dst_hbm, out_hbm, src_blk, dst_blk, rows,
            acc_sh, g0, g1, g2, g3, s0, s1, s2, s3, isem_s, isem_d):
    s = lax.axis_index("s")
    gsem = [g0, g1, g2, g3]
    ssem = [s0, s1, s2, s3]

    zeros16 = jnp.zeros((16,), jnp.float32)

    def _zero_row(i, carry):
        for k in range(D // 16):
            rows[0, i, pl.ds(k * 16, 16)] = zeros16
        return carry

    lax.fori_loop(0, CB, _zero_row, 0)
    for t in range(RPT // CB):
        pltpu.sync_copy(rows.at[0], acc_sh.at[pl.ds(s * RPT + t * CB, CB)])

    def _gather(p, jl, b):
        pltpu.async_copy(h_hbm.at[src_blk.at[p, jl]], rows.at[b], gsem[b])

    def _wait_gather(b):
        pltpu.make_async_copy(h_hbm.at[src_blk.at[0, 0]], rows.at[b],
                              gsem[b]).wait()

    def _scatter(p, jl, b):
        pltpu.async_copy(rows.at[b], acc_sh.at[dst_blk.at[p, jl]], ssem[b],
                         add=True)

    def _wait_scatter(b):
        pltpu.make_async_copy(rows.at[b], acc_sh.at[dst_blk.at[0, 0]],
                              ssem[b]).wait()

    pltpu.sync_copy(src_hbm.at[s, pl.ds(0, BK)], src_blk.at[0])
    pltpu.sync_copy(dst_hbm.at[s, pl.ds(0, BK)], dst_blk.at[0])
    plsc.subcore_barrier()

    def _block(kb, carry):
        p = lax.rem(kb, 2)

        @pl.when(kb > 0)
        def _():
            for b in range(NR):
                _wait_scatter(b)

        @pl.when(kb + 1 < NB)
        def _():
            pltpu.async_copy(src_hbm.at[s, pl.ds((kb + 1) * BK, BK)],
                             src_blk.at[1 - p], isem_s)
            pltpu.async_copy(dst_hbm.at[s, pl.ds((kb + 1) * BK, BK)],
                             dst_blk.at[1 - p], isem_d)

        for b in range(NR):
            _gather(p, b, b)
        for jl in range(BK):
            b = jl % NR
            _wait_gather(b)
            _scatter(p, jl, b)
            if 0 < jl and jl + NR - 1 < BK:
                bb = (jl - 1) % NR
                _wait_scatter(bb)
                _gather(p, jl + NR - 1, bb)

        @pl.when(kb + 1 < NB)
        def _():
            pltpu.make_async_copy(src_hbm.at[s, pl.ds(0, BK)],
                                  src_blk.at[0], isem_s).wait()
            pltpu.make_async_copy(dst_hbm.at[s, pl.ds(0, BK)],
                                  dst_blk.at[0], isem_d).wait()

        return carry

    lax.fori_loop(0, NB, _block, 0)
    for b in range(NR):
        _wait_scatter(b)

    plsc.subcore_barrier()
    pltpu.sync_copy(acc_sh.at[pl.ds(s * RPT, RPT)],
                    out_hbm.at[pl.ds(s * RPT, RPT)])


def _mlp_body(h_ref, agg_ref, wa_ref, ba_ref, ga_ref, bea_ref,
              wb_ref, bb_ref, gb_ref, beb_ref, batch_ref,
              h_out, pool_out):
    z = h_ref[...] + agg_ref[:N, :]

    def _lin_bn_relu(v, w_ref, b_ref, g_ref, be_ref):
        y = jnp.dot(v, w_ref[...], preferred_element_type=jnp.float32)
        y = y + b_ref[...]
        m = jnp.mean(y, axis=0, keepdims=True)
        var = jnp.mean(y * y, axis=0, keepdims=True) - m * m
        y = g_ref[...] * (y - m) * lax.rsqrt(var + 1e-5) + be_ref[...]
        return jnp.maximum(y, 0.0)

    y = _lin_bn_relu(z, wa_ref, ba_ref, ga_ref, bea_ref)
    y = _lin_bn_relu(y, wb_ref, bb_ref, gb_ref, beb_ref)
    h_out[...] = y

    seg = lax.broadcasted_iota(jnp.int32, (G, N), 0)
    onehot = jnp.where(seg == batch_ref[...], 1.0, 0.0)
    pool_out[...] = jnp.dot(onehot, y, preferred_element_type=jnp.float32)


_mlp = pl.pallas_call(
    _mlp_body,
    out_shape=(jax.ShapeDtypeStruct((N, D), jnp.float32),
               jax.ShapeDtypeStruct((G, D), jnp.float32)),
)


def _head_body(x_ref, batch_ref, p1, p2, p3, p4, p5,
               fc1w_ref, fc1b_ref, fc3w_ref, fc3b_ref, out_ref):
    seg = lax.broadcasted_iota(jnp.int32, (G, N), 0)
    onehot = jnp.where(seg == batch_ref[...], 1.0, 0.0)
    px = jnp.dot(onehot, x_ref[...], preferred_element_type=jnp.float32)
    hg = jnp.concatenate(
        [px, p1[...], p2[...], p3[...], p4[...], p5[...]], axis=1)
    r = jnp.dot(hg, fc1w_ref[...], preferred_element_type=jnp.float32)
    r = jnp.maximum(r + fc1b_ref[...], 0.0)
    o = jnp.dot(r, fc3w_ref[...], preferred_element_type=jnp.float32)
    out_ref[...] = o + fc3b_ref[...]


_head = pl.pallas_call(
    _head_body,
    out_shape=jax.ShapeDtypeStruct((G, OUT), jnp.float32),
)


def kernel(x, edge_index, batch, Wa, ba, ga, bea, Wb, bb, gb, beb,
           fc1_W, fc1_b, fc3_W, fc3_b):
    src = edge_index[0].astype(jnp.int32)
    dst = edge_index[1].astype(jnp.int32)
    pad = EP - E
    pad_i = lax.iota(jnp.int32, pad)
    src_p = jnp.concatenate([src, pad_i % N])
    dst_p = jnp.concatenate([dst, N + pad_i % (NA - N)])
    src_r = src_p.reshape(NS, CH, CB)
    dst_r = dst_p.reshape(NS, CH, CB)
    batch_r = batch.astype(jnp.int32).reshape(1, N)

    h = x
    pooled = []
    for i in range(5):
        agg = _sc_agg(h, src_r, dst_r)
        h, p = _mlp(h, agg, Wa[i],
                    ba[i].reshape(1, D), ga[i].reshape(1, D),
                    bea[i].reshape(1, D), Wb[i],
                    bb[i].reshape(1, D), gb[i].reshape(1, D),
                    beb[i].reshape(1, D), batch_r)
        pooled.append(p)

    return _head(x, batch_r, *pooled,
                 fc1_W, fc1_b.reshape(1, 6 * D), fc3_W, fc3_b.reshape(1, OUT))


# speedup vs baseline: 2.6507x; 1.0525x over previous
import functools

import jax
import jax.numpy as jnp
from jax import lax
from jax.experimental import pallas as pl
from jax.experimental.pallas import tpu as pltpu
from jax.experimental.pallas import tpu_sc as plsc

N = 10000
E = 320000
D = 128
G = 64
OUT = 16

NS = 16
CB = 32
NR = 8
BK = 32
NB = 20
CH = BK * NB
EP = NS * CH * CB
NA = 10240
RPT = NA // NS

_mesh = plsc.VectorSubcoreMesh(core_axis_name="c", subcore_axis_name="s",
                               num_cores=1)


@functools.partial(
    pl.kernel,
    mesh=_mesh,
    out_type=jax.ShapeDtypeStruct((NA, D), jnp.float32),
    scratch_types=[
        pltpu.VMEM((2, BK, CB), jnp.int32),
        pltpu.VMEM((2, BK, CB), jnp.int32),
        pltpu.VMEM((NR, CB, D), jnp.float32),
        pltpu.VMEM_SHARED((NA, D), jnp.float32),
    ] + [pltpu.SemaphoreType.DMA] * (2 * NR + 2),
)
def _sc_agg(h_hbm, src_hbm, dst_hbm, out_hbm, src_blk, dst_blk, rows,
            acc_sh, *sems):
    s = lax.axis_index("s")
    gsem = list(sems[:NR])
    ssem = list(sems[NR:2 * NR])
    isem_s = sems[2 * NR]
    isem_d = sems[2 * NR + 1]

    zeros16 = jnp.zeros((16,), jnp.float32)

    def _zero_row(i, carry):
        for k in range(D // 16):
            rows[0, i, pl.ds(k * 16, 16)] = zeros16
        return carry

    lax.fori_loop(0, CB, _zero_row, 0)
    for t in range(RPT // CB):
        pltpu.sync_copy(rows.at[0], acc_sh.at[pl.ds(s * RPT + t * CB, CB)])

    def _gather(p, jl, b):
        pltpu.async_copy(h_hbm.at[src_blk.at[p, jl]], rows.at[b], gsem[b])

    def _wait_gather(b):
        pltpu.make_async_copy(h_hbm.at[src_blk.at[0, 0]], rows.at[b],
                              gsem[b]).wait()

    def _scatter(p, jl, b):
        pltpu.async_copy(rows.at[b], acc_sh.at[dst_blk.at[p, jl]], ssem[b],
                         add=True)

    def _wait_scatter(b):
        pltpu.make_async_copy(rows.at[b], acc_sh.at[dst_blk.at[0, 0]],
                              ssem[b]).wait()

    pltpu.sync_copy(src_hbm.at[s, pl.ds(0, BK)], src_blk.at[0])
    pltpu.sync_copy(dst_hbm.at[s, pl.ds(0, BK)], dst_blk.at[0])
    plsc.subcore_barrier()

    def _block(kb, carry):
        p = lax.rem(kb, 2)

        @pl.when(kb > 0)
        def _():
            for b in range(NR):
                _wait_scatter(b)

        @pl.when(kb + 1 < NB)
        def _():
            pltpu.async_copy(src_hbm.at[s, pl.ds((kb + 1) * BK, BK)],
                             src_blk.at[1 - p], isem_s)
            pltpu.async_copy(dst_hbm.at[s, pl.ds((kb + 1) * BK, BK)],
                             dst_blk.at[1 - p], isem_d)

        for b in range(NR):
            _gather(p, b, b)
        for jl in range(BK):
            b = jl % NR
            _wait_gather(b)
            _scatter(p, jl, b)
            if 0 < jl and jl + NR - 1 < BK:
                bb = (jl - 1) % NR
                _wait_scatter(bb)
                _gather(p, jl + NR - 1, bb)

        @pl.when(kb + 1 < NB)
        def _():
            pltpu.make_async_copy(src_hbm.at[s, pl.ds(0, BK)],
                                  src_blk.at[0], isem_s).wait()
            pltpu.make_async_copy(dst_hbm.at[s, pl.ds(0, BK)],
                                  dst_blk.at[0], isem_d).wait()

        return carry

    lax.fori_loop(0, NB, _block, 0)
    for b in range(NR):
        _wait_scatter(b)

    plsc.subcore_barrier()
    pltpu.sync_copy(acc_sh.at[pl.ds(s * RPT, RPT)],
                    out_hbm.at[pl.ds(s * RPT, RPT)])


def _mlp_body(h_ref, agg_ref, wa_ref, ba_ref, ga_ref, bea_ref,
              wb_ref, bb_ref, gb_ref, beb_ref, batch_ref,
              h_out, pool_out):
    z = h_ref[...] + agg_ref[:N, :]

    def _lin_bn_relu(v, w_ref, b_ref, g_ref, be_ref):
        y = jnp.dot(v, w_ref[...], preferred_element_type=jnp.float32)
        y = y + b_ref[...]
        m = jnp.mean(y, axis=0, keepdims=True)
        var = jnp.mean(y * y, axis=0, keepdims=True) - m * m
        y = g_ref[...] * (y - m) * lax.rsqrt(var + 1e-5) + be_ref[...]
        return jnp.maximum(y, 0.0)

    y = _lin_bn_relu(z, wa_ref, ba_ref, ga_ref, bea_ref)
    y = _lin_bn_relu(y, wb_ref, bb_ref, gb_ref, beb_ref)
    h_out[...] = y

    seg = lax.broadcasted_iota(jnp.int32, (G, N), 0)
    onehot = jnp.where(seg == batch_ref[...], 1.0, 0.0)
    pool_out[...] = jnp.dot(onehot, y, preferred_element_type=jnp.float32)


_mlp = pl.pallas_call(
    _mlp_body,
    out_shape=(jax.ShapeDtypeStruct((N, D), jnp.float32),
               jax.ShapeDtypeStruct((G, D), jnp.float32)),
)


def _head_body(x_ref, batch_ref, p1, p2, p3, p4, p5,
               fc1w_ref, fc1b_ref, fc3w_ref, fc3b_ref, out_ref):
    seg = lax.broadcasted_iota(jnp.int32, (G, N), 0)
    onehot = jnp.where(seg == batch_ref[...], 1.0, 0.0)
    px = jnp.dot(onehot, x_ref[...], preferred_element_type=jnp.float32)
    hg = jnp.concatenate(
        [px, p1[...], p2[...], p3[...], p4[...], p5[...]], axis=1)
    r = jnp.dot(hg, fc1w_ref[...], preferred_element_type=jnp.float32)
    r = jnp.maximum(r + fc1b_ref[...], 0.0)
    o = jnp.dot(r, fc3w_ref[...], preferred_element_type=jnp.float32)
    out_ref[...] = o + fc3b_ref[...]


_head = pl.pallas_call(
    _head_body,
    out_shape=jax.ShapeDtypeStruct((G, OUT), jnp.float32),
)


def kernel(x, edge_index, batch, Wa, ba, ga, bea, Wb, bb, gb, beb,
           fc1_W, fc1_b, fc3_W, fc3_b):
    src = edge_index[0].astype(jnp.int32)
    dst = edge_index[1].astype(jnp.int32)
    pad = EP - E
    pad_i = lax.iota(jnp.int32, pad)
    src_p = jnp.concatenate([src, pad_i % N])
    dst_p = jnp.concatenate([dst, N + pad_i % (NA - N)])
    src_r = src_p.reshape(NS, CH, CB)
    dst_r = dst_p.reshape(NS, CH, CB)
    batch_r = batch.astype(jnp.int32).reshape(1, N)

    h = x
    pooled = []
    for i in range(5):
        agg = _sc_agg(h, src_r, dst_r)
        h, p = _mlp(h, agg, Wa[i],
                    ba[i].reshape(1, D), ga[i].reshape(1, D),
                    bea[i].reshape(1, D), Wb[i],
                    bb[i].reshape(1, D), gb[i].reshape(1, D),
                    beb[i].reshape(1, D), batch_r)
        pooled.append(p)

    return _head(x, batch_r, *pooled,
                 fc1_W, fc1_b.reshape(1, 6 * D), fc3_W, fc3_b.reshape(1, OUT))
